# baseline (speedup 1.0000x reference)
import functools

import jax
import jax.numpy as jnp
from jax import lax
from jax.experimental import pallas as pl
from jax.experimental.pallas import tpu as pltpu
from jax.experimental.pallas import tpu_sc as plsc

_B = 4
_SEQ = 8192
_L = 16
_NV = _SEQ // _L


def _make_mask_kernel():
    nc = 1

    mesh = plsc.VectorSubcoreMesh(
        core_axis_name="c", subcore_axis_name="s", num_cores=nc
    )

    @functools.partial(
        pl.kernel,
        mesh=mesh,
        out_type=jax.ShapeDtypeStruct((_B, _SEQ), jnp.int32),
        scratch_types=[
            pltpu.VMEM((_SEQ,), jnp.int32),
            pltpu.VMEM((_SEQ,), jnp.int32),
            pltpu.VMEM((_L,), jnp.int32),
        ],
        compiler_params=pltpu.CompilerParams(needs_layout_passes=False),
    )
    def mask_kernel(orders_hbm, thresh_hbm, out_hbm, row_v, outrow_v, th_v):
        c = lax.axis_index("c")
        s = lax.axis_index("s")
        wid = s * nc + c

        @pl.when(wid < _B)
        def _():
            pltpu.sync_copy(orders_hbm.at[wid], row_v)
            pltpu.sync_copy(thresh_hbm, th_v)
            lane = jnp.arange(_L, dtype=jnp.int32)
            t_thresh = jnp.sum(jnp.where(lane == wid, th_v[...], 0))

            def count_le(mid):
                @plsc.parallel_loop(
                    0, _NV, 1, unroll=16, carry=jnp.zeros((_L,), jnp.int32)
                )
                def acc(i, a):
                    x = row_v[pl.ds(i * _L, _L)]
                    return a + jnp.where(x <= mid, 1, 0)

                return jnp.sum(acc)

            def bs_body(_, carry):
                lo, hi, cless = carry
                mid = (lo + hi) // 2
                cnt = count_le(mid)
                ge = cnt >= t_thresh
                return (
                    jnp.where(ge, lo, mid + 1),
                    jnp.where(ge, mid, hi),
                    jnp.where(ge, cless, cnt),
                )

            vstar, _, count_less = lax.fori_loop(
                0,
                13,
                bs_body,
                (jnp.int32(0), jnp.int32(_SEQ - 1), jnp.int32(0)),
            )
            num_ties = t_thresh - count_less

            @plsc.parallel_loop(0, _NV, 1, unroll=8, carry=jnp.int32(0))
            def _mask_loop(i, tie_off):
                x = row_v[pl.ds(i * _L, _L)]
                lt = x < vstar
                eq = x == vstar
                eq_i = jnp.where(eq, 1, 0)
                incl = plsc.cumsum(eq_i)
                tie_rank = tie_off + incl - eq_i
                m = lt | (eq & (tie_rank < num_ties))
                outrow_v[pl.ds(i * _L, _L)] = jnp.where(m, 1, 0)
                return tie_off + jnp.sum(eq_i)
            pltpu.sync_copy(outrow_v, out_hbm.at[wid])

    return mask_kernel


_mask_kernel = _make_mask_kernel()


def kernel(patches, orders):
    batch_size, seq_len, _hidden = patches.shape
    std = 0.25
    mean = 1.0
    a, b = 0.0, 1.0
    rkey = jax.random.key(42)
    lower = (a - mean) / std
    upper = (b - mean) / std
    mask_rates = (
        jax.random.truncated_normal(rkey, lower, upper, (batch_size,), jnp.float32)
        * std
        + mean
    )
    thresholds = jnp.ceil(mask_rates * seq_len).astype(jnp.int32)
    th_padded = jnp.zeros((_L,), jnp.int32).at[:batch_size].set(thresholds)
    out_i32 = _mask_kernel(orders.astype(jnp.int32), th_padded)
    return out_i32.astype(jnp.bool_)


# speedup vs baseline: 1.0494x; 1.0494x over previous
import functools

import jax
import jax.numpy as jnp
from jax import lax
from jax.experimental import pallas as pl
from jax.experimental.pallas import tpu as pltpu
from jax.experimental.pallas import tpu_sc as plsc

_B = 4
_SEQ = 8192
_L = 16
_NV = _SEQ // _L


def _make_mask_kernel():
    nc = 1

    mesh = plsc.VectorSubcoreMesh(
        core_axis_name="c", subcore_axis_name="s", num_cores=nc
    )

    @functools.partial(
        pl.kernel,
        mesh=mesh,
        out_type=jax.ShapeDtypeStruct((_B, _SEQ), jnp.int32),
        scratch_types=[
            pltpu.VMEM((_SEQ,), jnp.int32),
            pltpu.VMEM((_SEQ,), jnp.int32),
            pltpu.VMEM((_L,), jnp.int32),
        ],
        compiler_params=pltpu.CompilerParams(needs_layout_passes=False),
    )
    def mask_kernel(orders_hbm, thresh_hbm, out_hbm, row_v, outrow_v, th_v):
        c = lax.axis_index("c")
        s = lax.axis_index("s")
        wid = s * nc + c

        @pl.when(wid < _B)
        def _():
            pltpu.sync_copy(orders_hbm.at[wid], row_v)
            pltpu.sync_copy(thresh_hbm, th_v)
            lane = jnp.arange(_L, dtype=jnp.int32)
            t_thresh = jnp.sum(jnp.where(lane == wid, th_v[...], 0))

            def count_le(mid):
                @plsc.parallel_loop(
                    0, _NV, 1, unroll=8, carry=jnp.zeros((_L,), jnp.int32)
                )
                def acc(i, a):
                    x = row_v[pl.ds(i * _L, _L)]
                    return a + jnp.where(x <= mid, 1, 0)

                return jnp.sum(acc)

            def bs_body(_, carry):
                lo, hi, cless = carry
                mid = (lo + hi) // 2
                cnt = count_le(mid)
                ge = cnt >= t_thresh
                return (
                    jnp.where(ge, lo, mid + 1),
                    jnp.where(ge, mid, hi),
                    jnp.where(ge, cless, cnt),
                )

            vstar, _, count_less = lax.fori_loop(
                0,
                13,
                bs_body,
                (jnp.int32(0), jnp.int32(_SEQ - 1), jnp.int32(0)),
            )
            num_ties = t_thresh - count_less

            @plsc.parallel_loop(
                0, _NV, 1, unroll=4, carry=jnp.zeros((_L,), jnp.int32)
            )
            def _mask_loop(i, tie_off):
                x = row_v[pl.ds(i * _L, _L)]
                lt = x < vstar
                eq = x == vstar
                eq_i = jnp.where(eq, 1, 0)
                incl = plsc.cumsum(eq_i)
                tie_rank = tie_off + incl - eq_i
                m = lt | (eq & (tie_rank < num_ties))
                outrow_v[pl.ds(i * _L, _L)] = jnp.where(m, 1, 0)
                return tie_off + plsc.all_reduce_population_count(eq)
            pltpu.sync_copy(outrow_v, out_hbm.at[wid])

    return mask_kernel


_mask_kernel = _make_mask_kernel()


def kernel(patches, orders):
    batch_size, seq_len, _hidden = patches.shape
    std = 0.25
    mean = 1.0
    a, b = 0.0, 1.0
    rkey = jax.random.key(42)
    lower = (a - mean) / std
    upper = (b - mean) / std
    mask_rates = (
        jax.random.truncated_normal(rkey, lower, upper, (batch_size,), jnp.float32)
        * std
        + mean
    )
    thresholds = jnp.ceil(mask_rates * seq_len).astype(jnp.int32)
    th_padded = jnp.zeros((_L,), jnp.int32).at[:batch_size].set(thresholds)
    out_i32 = _mask_kernel(orders.astype(jnp.int32), th_padded)
    return out_i32.astype(jnp.bool_)
